# baseline (speedup 1.0000x reference)
import functools

import jax
import jax.numpy as jnp
from jax import lax
from jax.experimental import pallas as pl
from jax.experimental.pallas import tpu as pltpu
from jax.experimental.pallas import tpu_sc as plsc

N = 10000
S = 32
D = 128
B = 1024

NC = 2
NS = 16
NW = NC * NS
L = 16
DJ = D // L

NODES_PW1 = 320
GN = 4
ROWS_G = GN * S
NG1 = NODES_PW1 // GN
NG_TAIL = (N - (NW - 1) * NODES_PW1) // GN

NB2 = B // NW

def _accum_mean(buf, row0, out_ref, out_row):
    def body(r, acc):
        return tuple(acc[j] + buf[row0 + r, pl.ds(j * L, L)] for j in range(DJ))

    acc0 = tuple(jnp.zeros((L,), jnp.float32) for _ in range(DJ))
    acc = lax.fori_loop(0, S, body, acc0)
    inv = jnp.float32(1.0 / S)
    for j in range(DJ):
        out_ref[out_row, pl.ds(j * L, L)] = acc[j] * inv


def _sc_mean_all_body(x_hbm, nbr_hbm, out_hbm, xs_shr, idx_v, buf0, buf1,
                      ost0, ost1, sem0, sem1, osem0, osem1):
    sid = lax.axis_index("s")
    wid = sid * NC + lax.axis_index("c")
    base = wid * NODES_PW1
    @pl.when(sid < NS - 1)
    def _():
        pltpu.sync_copy(x_hbm.at[pl.ds(sid * 624, 624), :],
                        xs_shr.at[pl.ds(sid * 624, 624), :])

    @pl.when(sid == NS - 1)
    def _():
        pltpu.sync_copy(x_hbm.at[pl.ds((NS - 1) * 624, N - (NS - 1) * 624), :],
                        xs_shr.at[pl.ds((NS - 1) * 624, N - (NS - 1) * 624), :])
    @pl.when(wid < NW - 1)
    def _():
        pltpu.sync_copy(nbr_hbm.at[pl.ds(wid * NG1, NG1), :], idx_v)

    @pl.when(wid == NW - 1)
    def _():
        pltpu.sync_copy(nbr_hbm.at[pl.ds((NW - 1) * NG1, NG_TAIL), :],
                        idx_v.at[pl.ds(0, NG_TAIL), :])
    plsc.subcore_barrier()
    ng = jnp.where(wid == NW - 1, NG_TAIL, NG1)
    nq = ng // 4

    bufs = (buf0, buf1)
    sems = (sem0, sem1)
    osts = (ost0, ost1)
    osems = (osem0, osem1)
    pltpu.async_copy(xs_shr.at[idx_v.at[0]], buf0, sem0)
    pltpu.async_copy(xs_shr.at[idx_v.at[1]], buf1, sem1)

    def owrite(q, p):
        return pltpu.make_async_copy(
            osts[p], out_hbm.at[pl.ds(base + q * 16 + p * 8, 8), :],
            osems[p])

    def quad(q, carry):
        for j in range(4):
            g = q * 4 + j
            b = j % 2
            p = j // 2
            if j % 2 == 0:
                @pl.when(q > 0)
                def _():
                    owrite(q - 1, p).wait()
            pltpu.make_async_copy(xs_shr.at[idx_v.at[g]], bufs[b],
                                  sems[b]).wait()
            for n in range(GN):
                _accum_mean(bufs[b], n * S, osts[p], (j % 2) * GN + n)

            @pl.when(g + 2 < ng)
            def _():
                pltpu.async_copy(xs_shr.at[idx_v.at[g + 2]], bufs[b], sems[b])
            if j % 2 == 1:
                pltpu.async_copy(
                    osts[p],
                    out_hbm.at[pl.ds(base + q * 16 + p * 8, 8), :], osems[p])
        return carry

    lax.fori_loop(0, nq, quad, 0)
    owrite(nq - 1, 0).wait()
    owrite(nq - 1, 1).wait()


def _sc_layer2_body(h1_hbm, nbr_hbm, bidx_hbm, hb_hbm, hn_hbm,
                    h1s, bidx_v, nbr_v, idx_c, buf0, buf1, outb, hb_v,
                    sem_hb, sem0, sem1):
    sid = lax.axis_index("s")
    wid = sid * NC + lax.axis_index("c")
    base = wid * NB2

    @pl.when(sid < NS - 1)
    def _():
        pltpu.sync_copy(h1_hbm.at[pl.ds(sid * 624, 624), :],
                        h1s.at[pl.ds(sid * 624, 624), :])

    @pl.when(sid == NS - 1)
    def _():
        pltpu.sync_copy(h1_hbm.at[pl.ds((NS - 1) * 624, N - (NS - 1) * 624), :],
                        h1s.at[pl.ds((NS - 1) * 624, N - (NS - 1) * 624), :])
    pltpu.sync_copy(bidx_hbm.at[pl.ds(base, NB2)], bidx_v)
    pltpu.sync_copy(nbr_hbm.at[bidx_v], nbr_v)
    plsc.subcore_barrier()
    pltpu.async_copy(h1s.at[bidx_v], hb_v, sem_hb)

    for n in range(NB2):
        for h in range(2):
            pos = n * S + h * L
            idx_c[pos // ROWS_G, pl.ds(pos % ROWS_G, L)] = \
                nbr_v[n, pl.ds(h * L, L)]

    bufs = (buf0, buf1)
    sems = (sem0, sem1)
    NG2 = NB2 * S // ROWS_G
    pltpu.async_copy(h1s.at[idx_c.at[0]], buf0, sem0)
    pltpu.async_copy(h1s.at[idx_c.at[1]], buf1, sem1)

    def pair(gg, carry):
        for b in range(2):
            g = gg * 2 + b
            pltpu.make_async_copy(h1s.at[idx_c.at[g]], bufs[b],
                                  sems[b]).wait()
            for n in range(GN):
                _accum_mean(bufs[b], n * S, outb, g * GN + n)

            @pl.when(g + 2 < NG2)
            def _():
                pltpu.async_copy(h1s.at[idx_c.at[g + 2]], bufs[b], sems[b])
        return carry

    lax.fori_loop(0, NG2 // 2, pair, 0)

    pltpu.make_async_copy(h1s.at[bidx_v], hb_v, sem_hb).wait()
    pltpu.sync_copy(hb_v, hb_hbm.at[pl.ds(base, NB2), :])
    pltpu.sync_copy(outb, hn_hbm.at[pl.ds(base, NB2), :])


def _tc_dense_body(x_ref, hn_ref, wa_ref, wb_ref, b_ref, o_ref):
    h = jnp.dot(x_ref[...], wa_ref[...], preferred_element_type=jnp.float32)
    h = h + jnp.dot(hn_ref[...], wb_ref[...],
                    preferred_element_type=jnp.float32)
    h = jnp.maximum(h + b_ref[...], 0.0)
    nrm = jnp.sqrt(jnp.sum(h * h, axis=1, keepdims=True))
    o_ref[...] = h / jnp.where(nrm == 0.0, 1.0, nrm)


def _tc_dense(x, hn, wa, wb, b, rows, block):
    grid = rows // block
    bs = lambda: pl.BlockSpec((block, D), lambda i: (i, 0))
    ws = lambda: pl.BlockSpec((D, D), lambda i: (0, 0))
    return pl.pallas_call(
        _tc_dense_body,
        grid=(grid,),
        in_specs=[bs(), bs(), ws(), ws(),
                  pl.BlockSpec((1, D), lambda i: (0, 0))],
        out_specs=bs(),
        out_shape=jax.ShapeDtypeStruct((rows, D), jnp.float32),
    )(x, hn, wa, wb, b)


@functools.cache
def _sc_kernels():
    mesh = plsc.VectorSubcoreMesh(core_axis_name="c", subcore_axis_name="s",
                                  num_cores=NC, num_subcores=NS)
    sc_mean_all = pl.kernel(
        _sc_mean_all_body,
        out_type=jax.ShapeDtypeStruct((N, D), jnp.float32),
        mesh=mesh,
        scratch_types=[
            pltpu.VMEM_SHARED((N, D), jnp.float32),
            pltpu.VMEM((NG1, ROWS_G), jnp.int32),
            pltpu.VMEM((ROWS_G, D), jnp.float32),
            pltpu.VMEM((ROWS_G, D), jnp.float32),
            pltpu.VMEM((8, D), jnp.float32),
            pltpu.VMEM((8, D), jnp.float32),
            pltpu.SemaphoreType.DMA,
            pltpu.SemaphoreType.DMA,
            pltpu.SemaphoreType.DMA,
            pltpu.SemaphoreType.DMA,
        ],
    )
    sc_layer2 = pl.kernel(
        _sc_layer2_body,
        out_type=(jax.ShapeDtypeStruct((B, D), jnp.float32),
                  jax.ShapeDtypeStruct((B, D), jnp.float32)),
        mesh=mesh,
        scratch_types=[
            pltpu.VMEM_SHARED((N, D), jnp.float32),
            pltpu.VMEM((NB2,), jnp.int32),
            pltpu.VMEM((NB2, 128), jnp.int32),
            pltpu.VMEM((NB2 * S // ROWS_G, ROWS_G), jnp.int32),
            pltpu.VMEM((ROWS_G, D), jnp.float32),
            pltpu.VMEM((ROWS_G, D), jnp.float32),
            pltpu.VMEM((NB2, D), jnp.float32),
            pltpu.VMEM((NB2, D), jnp.float32),
            pltpu.SemaphoreType.DMA,
            pltpu.SemaphoreType.DMA,
            pltpu.SemaphoreType.DMA,
        ],
    )
    return sc_mean_all, sc_layer2


@jax.jit
def kernel(x, nbr0, nbr1, batch_nodes, W0, b0, W1, b1):
    nbr0_rows = nbr0.reshape(N * S // ROWS_G, ROWS_G)

    nbr1_pad = jnp.pad(nbr1, ((0, 0), (0, 128 - S)))


    sc_mean_all, sc_layer2 = _sc_kernels()
    hn0 = sc_mean_all(x, nbr0_rows)
    h1 = _tc_dense(x, hn0, W0[:D], W0[D:], b0.reshape(1, D),
                   rows=N, block=2000)
    hb, hn1 = sc_layer2(h1, nbr1_pad, batch_nodes)
    z = _tc_dense(hb, hn1, W1[:D], W1[D:], b1.reshape(1, D),
                  rows=B, block=1024)
    return z


# speedup vs baseline: 1.0180x; 1.0180x over previous
import functools

import jax
import jax.numpy as jnp
from jax import lax
from jax.experimental import pallas as pl
from jax.experimental.pallas import tpu as pltpu
from jax.experimental.pallas import tpu_sc as plsc

N = 10000
S = 32
D = 128
B = 1024

NC = 2
NS = 16
NW = NC * NS
L = 16
DJ = D // L

NODES_PW1 = 320
GN = 4
ROWS_G = GN * S
NG1 = NODES_PW1 // GN
NG_TAIL = (N - (NW - 1) * NODES_PW1) // GN

NB2 = B // NW

def _accum_mean(buf, row0, out_ref, out_row):
    def body(r, acc):
        return tuple(acc[j] + buf[row0 + r, pl.ds(j * L, L)] for j in range(DJ))

    acc0 = tuple(jnp.zeros((L,), jnp.float32) for _ in range(DJ))
    acc = lax.fori_loop(0, S, body, acc0)
    inv = jnp.float32(1.0 / S)
    for j in range(DJ):
        out_ref[out_row, pl.ds(j * L, L)] = acc[j] * inv


def _sc_mean_all_body(x_hbm, nbr_hbm, out_hbm, xs_shr, idx_v, buf0, buf1,
                      ost0, ost1, sem0, sem1, osem0, osem1):
    sid = lax.axis_index("s")
    wid = sid * NC + lax.axis_index("c")
    base = wid * NODES_PW1
    @pl.when(sid < NS - 1)
    def _():
        pltpu.async_copy(x_hbm.at[pl.ds(sid * 624, 624), :],
                         xs_shr.at[pl.ds(sid * 624, 624), :], osem0)

    @pl.when(sid == NS - 1)
    def _():
        pltpu.async_copy(x_hbm.at[pl.ds((NS - 1) * 624, N - (NS - 1) * 624), :],
                         xs_shr.at[pl.ds((NS - 1) * 624, N - (NS - 1) * 624), :],
                         osem0)
    @pl.when(wid < NW - 1)
    def _():
        pltpu.sync_copy(nbr_hbm.at[pl.ds(wid * NG1, NG1), :], idx_v)

    @pl.when(wid == NW - 1)
    def _():
        pltpu.sync_copy(nbr_hbm.at[pl.ds((NW - 1) * NG1, NG_TAIL), :],
                        idx_v.at[pl.ds(0, NG_TAIL), :])

    @pl.when(sid < NS - 1)
    def _():
        pltpu.make_async_copy(x_hbm.at[pl.ds(sid * 624, 624), :],
                              xs_shr.at[pl.ds(sid * 624, 624), :],
                              osem0).wait()

    @pl.when(sid == NS - 1)
    def _():
        pltpu.make_async_copy(
            x_hbm.at[pl.ds((NS - 1) * 624, N - (NS - 1) * 624), :],
            xs_shr.at[pl.ds((NS - 1) * 624, N - (NS - 1) * 624), :],
            osem0).wait()
    plsc.subcore_barrier()
    ng = jnp.where(wid == NW - 1, NG_TAIL, NG1)
    nq = ng // 4

    bufs = (buf0, buf1)
    sems = (sem0, sem1)
    osts = (ost0, ost1)
    osems = (osem0, osem1)
    pltpu.async_copy(xs_shr.at[idx_v.at[0]], buf0, sem0)
    pltpu.async_copy(xs_shr.at[idx_v.at[1]], buf1, sem1)

    def owrite(q, p):
        return pltpu.make_async_copy(
            osts[p], out_hbm.at[pl.ds(base + q * 16 + p * 8, 8), :],
            osems[p])

    def quad(q, carry):
        for j in range(4):
            g = q * 4 + j
            b = j % 2
            p = j // 2
            if j % 2 == 0:
                @pl.when(q > 0)
                def _():
                    owrite(q - 1, p).wait()
            pltpu.make_async_copy(xs_shr.at[idx_v.at[g]], bufs[b],
                                  sems[b]).wait()
            for n in range(GN):
                _accum_mean(bufs[b], n * S, osts[p], (j % 2) * GN + n)

            @pl.when(g + 2 < ng)
            def _():
                pltpu.async_copy(xs_shr.at[idx_v.at[g + 2]], bufs[b], sems[b])
            if j % 2 == 1:
                pltpu.async_copy(
                    osts[p],
                    out_hbm.at[pl.ds(base + q * 16 + p * 8, 8), :], osems[p])
        return carry

    lax.fori_loop(0, nq, quad, 0)
    owrite(nq - 1, 0).wait()
    owrite(nq - 1, 1).wait()


def _sc_layer2_body(h1_hbm, nbr_hbm, bidx_hbm, hb_hbm, hn_hbm,
                    h1s, bidx_v, nbr_v, idx_c, buf0, buf1, outb, hb_v,
                    sem_hb, sem0, sem1):
    sid = lax.axis_index("s")
    wid = sid * NC + lax.axis_index("c")
    base = wid * NB2

    @pl.when(sid < NS - 1)
    def _():
        pltpu.async_copy(h1_hbm.at[pl.ds(sid * 624, 624), :],
                         h1s.at[pl.ds(sid * 624, 624), :], sem0)

    @pl.when(sid == NS - 1)
    def _():
        pltpu.async_copy(h1_hbm.at[pl.ds((NS - 1) * 624, N - (NS - 1) * 624), :],
                         h1s.at[pl.ds((NS - 1) * 624, N - (NS - 1) * 624), :],
                         sem0)
    pltpu.sync_copy(bidx_hbm.at[pl.ds(base, NB2)], bidx_v)
    pltpu.sync_copy(nbr_hbm.at[bidx_v], nbr_v)

    @pl.when(sid < NS - 1)
    def _():
        pltpu.make_async_copy(h1_hbm.at[pl.ds(sid * 624, 624), :],
                              h1s.at[pl.ds(sid * 624, 624), :], sem0).wait()

    @pl.when(sid == NS - 1)
    def _():
        pltpu.make_async_copy(
            h1_hbm.at[pl.ds((NS - 1) * 624, N - (NS - 1) * 624), :],
            h1s.at[pl.ds((NS - 1) * 624, N - (NS - 1) * 624), :], sem0).wait()
    plsc.subcore_barrier()
    pltpu.async_copy(h1s.at[bidx_v], hb_v, sem_hb)

    for n in range(NB2):
        for h in range(2):
            pos = n * S + h * L
            idx_c[pos // ROWS_G, pl.ds(pos % ROWS_G, L)] = \
                nbr_v[n, pl.ds(h * L, L)]

    bufs = (buf0, buf1)
    sems = (sem0, sem1)
    NG2 = NB2 * S // ROWS_G
    pltpu.async_copy(h1s.at[idx_c.at[0]], buf0, sem0)
    pltpu.async_copy(h1s.at[idx_c.at[1]], buf1, sem1)

    def pair(gg, carry):
        for b in range(2):
            g = gg * 2 + b
            pltpu.make_async_copy(h1s.at[idx_c.at[g]], bufs[b],
                                  sems[b]).wait()
            for n in range(GN):
                _accum_mean(bufs[b], n * S, outb, g * GN + n)

            @pl.when(g + 2 < NG2)
            def _():
                pltpu.async_copy(h1s.at[idx_c.at[g + 2]], bufs[b], sems[b])
        return carry

    lax.fori_loop(0, NG2 // 2, pair, 0)

    pltpu.make_async_copy(h1s.at[bidx_v], hb_v, sem_hb).wait()
    pltpu.sync_copy(hb_v, hb_hbm.at[pl.ds(base, NB2), :])
    pltpu.sync_copy(outb, hn_hbm.at[pl.ds(base, NB2), :])


def _tc_dense_body(x_ref, hn_ref, wa_ref, wb_ref, b_ref, o_ref):
    h = jnp.dot(x_ref[...], wa_ref[...], preferred_element_type=jnp.float32)
    h = h + jnp.dot(hn_ref[...], wb_ref[...],
                    preferred_element_type=jnp.float32)
    h = jnp.maximum(h + b_ref[...], 0.0)
    nrm = jnp.sqrt(jnp.sum(h * h, axis=1, keepdims=True))
    o_ref[...] = h / jnp.where(nrm == 0.0, 1.0, nrm)


def _tc_dense(x, hn, wa, wb, b, rows, block):
    grid = rows // block
    bs = lambda: pl.BlockSpec((block, D), lambda i: (i, 0))
    ws = lambda: pl.BlockSpec((D, D), lambda i: (0, 0))
    return pl.pallas_call(
        _tc_dense_body,
        grid=(grid,),
        in_specs=[bs(), bs(), ws(), ws(),
                  pl.BlockSpec((1, D), lambda i: (0, 0))],
        out_specs=bs(),
        out_shape=jax.ShapeDtypeStruct((rows, D), jnp.float32),
    )(x, hn, wa, wb, b)


@functools.cache
def _sc_kernels():
    mesh = plsc.VectorSubcoreMesh(core_axis_name="c", subcore_axis_name="s",
                                  num_cores=NC, num_subcores=NS)
    sc_mean_all = pl.kernel(
        _sc_mean_all_body,
        out_type=jax.ShapeDtypeStruct((N, D), jnp.float32),
        mesh=mesh,
        scratch_types=[
            pltpu.VMEM_SHARED((N, D), jnp.float32),
            pltpu.VMEM((NG1, ROWS_G), jnp.int32),
            pltpu.VMEM((ROWS_G, D), jnp.float32),
            pltpu.VMEM((ROWS_G, D), jnp.float32),
            pltpu.VMEM((8, D), jnp.float32),
            pltpu.VMEM((8, D), jnp.float32),
            pltpu.SemaphoreType.DMA,
            pltpu.SemaphoreType.DMA,
            pltpu.SemaphoreType.DMA,
            pltpu.SemaphoreType.DMA,
        ],
    )
    sc_layer2 = pl.kernel(
        _sc_layer2_body,
        out_type=(jax.ShapeDtypeStruct((B, D), jnp.float32),
                  jax.ShapeDtypeStruct((B, D), jnp.float32)),
        mesh=mesh,
        scratch_types=[
            pltpu.VMEM_SHARED((N, D), jnp.float32),
            pltpu.VMEM((NB2,), jnp.int32),
            pltpu.VMEM((NB2, 128), jnp.int32),
            pltpu.VMEM((NB2 * S // ROWS_G, ROWS_G), jnp.int32),
            pltpu.VMEM((ROWS_G, D), jnp.float32),
            pltpu.VMEM((ROWS_G, D), jnp.float32),
            pltpu.VMEM((NB2, D), jnp.float32),
            pltpu.VMEM((NB2, D), jnp.float32),
            pltpu.SemaphoreType.DMA,
            pltpu.SemaphoreType.DMA,
            pltpu.SemaphoreType.DMA,
        ],
    )
    return sc_mean_all, sc_layer2


@jax.jit
def kernel(x, nbr0, nbr1, batch_nodes, W0, b0, W1, b1):
    nbr0_rows = nbr0.reshape(N * S // ROWS_G, ROWS_G)

    nbr1_pad = jnp.pad(nbr1, ((0, 0), (0, 128 - S)))


    sc_mean_all, sc_layer2 = _sc_kernels()
    hn0 = sc_mean_all(x, nbr0_rows)
    h1 = _tc_dense(x, hn0, W0[:D], W0[D:], b0.reshape(1, D),
                   rows=N, block=2000)
    hb, hn1 = sc_layer2(h1, nbr1_pad, batch_nodes)
    z = _tc_dense(hb, hn1, W1[:D], W1[D:], b1.reshape(1, D),
                  rows=B, block=1024)
    return z
